# baseline (speedup 1.0000x reference)
import jax
import jax.numpy as jnp
import numpy as np
from jax.experimental import pallas as pl
from jax.experimental.pallas import tpu as pltpu

B = 128
V = 100000
VBLK = 2000
GRID = V // VBLK

_K0 = 0
_K1 = 42
_KS2 = (0x1BD11BDA ^ _K0 ^ _K1) & 0xFFFFFFFF
_TINY = np.float32(1.1754944e-38)
_LOG2E = np.float32(1.4426950408889634)
_LN2 = np.float32(0.6931471805599453)


def _rotl(x, r):
    return (x << jnp.uint32(r)) | (x >> jnp.uint32(32 - r))


def _four_rounds(x0, x1, rots):
    for r in rots:
        x0 = x0 + x1
        x1 = _rotl(x1, r)
        x1 = x1 ^ x0
    return x0, x1


def _threefry_bits(cnt_u32):
    k0 = jnp.uint32(_K0)
    k1 = jnp.uint32(_K1)
    ks2 = jnp.uint32(_KS2)
    r1 = (13, 15, 26, 6)
    r2 = (17, 29, 16, 24)
    x0 = jnp.zeros_like(cnt_u32) + k0
    x1 = cnt_u32 + k1
    x0, x1 = _four_rounds(x0, x1, r1)
    x0 = x0 + k1
    x1 = x1 + ks2 + jnp.uint32(1)
    x0, x1 = _four_rounds(x0, x1, r2)
    x0 = x0 + ks2
    x1 = x1 + k0 + jnp.uint32(2)
    x0, x1 = _four_rounds(x0, x1, r1)
    x0 = x0 + k0
    x1 = x1 + k1 + jnp.uint32(3)
    x0, x1 = _four_rounds(x0, x1, r2)
    x0 = x0 + k1
    x1 = x1 + ks2 + jnp.uint32(4)
    x0, x1 = _four_rounds(x0, x1, r1)
    x0 = x0 + ks2
    x1 = x1 + k0 + jnp.uint32(5)
    return x0 ^ x1


def _main_body(cnt_ref, st2_ref, w_ref, b2_ref, z_ref, i_ref, l_ref, s_ref):
    j = pl.program_id(0)

    l2 = jax.lax.dot_general(
        w_ref[...], st2_ref[...], (((1,), (0,)), ((), ())),
        preferred_element_type=jnp.float32)
    l2 = l2 + b2_ref[...]

    cnt = cnt_ref[...] + j * VBLK
    bits = _threefry_bits(jax.lax.bitcast_convert_type(cnt, jnp.uint32))
    fb = (bits >> jnp.uint32(9)) | jnp.uint32(0x3F800000)
    f = jax.lax.bitcast_convert_type(fb, jnp.float32) - jnp.float32(1.0)
    u = jnp.maximum(f, _TINY)
    nlu = jnp.log(u) * jnp.float32(-1.0)
    z = nlu * jnp.exp2(-l2)

    s_ref[...] = jnp.sum(jnp.exp2(l2), axis=0, keepdims=True)[None]

    blk_zmin = jnp.min(z, axis=0, keepdims=True)
    is_min = z == blk_zmin
    blk_cnt = jnp.min(jnp.where(is_min, cnt, jnp.int32(2**31 - 1)),
                      axis=0, keepdims=True)
    l2_at = jnp.max(jnp.where(is_min, l2, -jnp.inf), axis=0, keepdims=True)
    z_ref[...] = blk_zmin[None]
    i_ref[...] = blk_cnt[None]
    l_ref[...] = l2_at[None]


def _merge_body(z_ref, i_ref, l_ref, s_ref, a_ref, lp_ref):
    z = z_ref[...]
    zmin = jnp.min(z, axis=0, keepdims=True)
    is_min = z == zmin
    bi = jax.lax.broadcasted_iota(jnp.int32, (GRID, 1, B), 0)
    kblk = jnp.min(jnp.where(is_min, bi, GRID), axis=0, keepdims=True)
    sel = bi == kblk
    cnt_win = jnp.min(jnp.where(sel, i_ref[...], jnp.int32(2**31 - 1)), axis=0)
    rv = jax.lax.broadcasted_iota(jnp.int32, (1, B), 1) * V
    a_ref[...] = cnt_win - rv
    l2_a = jnp.max(jnp.where(sel, l_ref[...], -jnp.inf), axis=0)
    lp_ref[...] = (l2_a - jnp.log2(jnp.sum(s_ref[...], axis=0))) * _LN2


def kernel(s, W, b):
    st2 = s.T * _LOG2E
    b2 = b.reshape(V, 1) * _LOG2E
    vi = jax.lax.broadcasted_iota(jnp.int32, (VBLK, B), 0)
    ri = jax.lax.broadcasted_iota(jnp.int32, (VBLK, B), 1)
    cnt0 = ri * V + vi
    part_spec = pl.BlockSpec((1, 1, B), lambda j: (j, 0, 0))
    part_shape = jax.ShapeDtypeStruct((GRID, 1, B), jnp.float32)
    zp, ip, lp_part, sp = pl.pallas_call(
        _main_body,
        grid=(GRID,),
        in_specs=[
            pl.BlockSpec((VBLK, B), lambda j: (0, 0)),
            pl.BlockSpec((64, B), lambda j: (0, 0)),
            pl.BlockSpec((VBLK, 64), lambda j: (j, 0)),
            pl.BlockSpec((VBLK, 1), lambda j: (j, 0)),
        ],
        out_specs=[part_spec, part_spec, part_spec, part_spec],
        out_shape=[
            part_shape,
            jax.ShapeDtypeStruct((GRID, 1, B), jnp.int32),
            part_shape,
            part_shape,
        ],
        compiler_params=pltpu.CompilerParams(
            dimension_semantics=("parallel",)),
    )(cnt0, st2, W, b2)
    a2, lp2 = pl.pallas_call(
        _merge_body,
        out_shape=[
            jax.ShapeDtypeStruct((1, B), jnp.int32),
            jax.ShapeDtypeStruct((1, B), jnp.float32),
        ],
    )(zp, ip, lp_part, sp)
    return a2.reshape(B), lp2.reshape(B)


# speedup vs baseline: 1.0183x; 1.0183x over previous
import jax
import jax.numpy as jnp
import numpy as np
from jax.experimental import pallas as pl
from jax.experimental.pallas import tpu as pltpu

B = 128
V = 100000
VBLK = 5000
GRID = V // VBLK

_K0 = 0
_K1 = 42
_KS2 = (0x1BD11BDA ^ _K0 ^ _K1) & 0xFFFFFFFF
_TINY = np.float32(1.1754944e-38)
_LOG2E = np.float32(1.4426950408889634)
_LN2 = np.float32(0.6931471805599453)


def _rotl(x, r):
    return (x << jnp.uint32(r)) | (x >> jnp.uint32(32 - r))


def _four_rounds(x0, x1, rots):
    for r in rots:
        x0 = x0 + x1
        x1 = _rotl(x1, r)
        x1 = x1 ^ x0
    return x0, x1


def _threefry_bits(cnt_u32):
    k0 = jnp.uint32(_K0)
    k1 = jnp.uint32(_K1)
    ks2 = jnp.uint32(_KS2)
    r1 = (13, 15, 26, 6)
    r2 = (17, 29, 16, 24)
    x0 = jnp.zeros_like(cnt_u32) + k0
    x1 = cnt_u32 + k1
    x0, x1 = _four_rounds(x0, x1, r1)
    x0 = x0 + k1
    x1 = x1 + ks2 + jnp.uint32(1)
    x0, x1 = _four_rounds(x0, x1, r2)
    x0 = x0 + ks2
    x1 = x1 + k0 + jnp.uint32(2)
    x0, x1 = _four_rounds(x0, x1, r1)
    x0 = x0 + k0
    x1 = x1 + k1 + jnp.uint32(3)
    x0, x1 = _four_rounds(x0, x1, r2)
    x0 = x0 + k1
    x1 = x1 + ks2 + jnp.uint32(4)
    x0, x1 = _four_rounds(x0, x1, r1)
    x0 = x0 + ks2
    x1 = x1 + k0 + jnp.uint32(5)
    return x0 ^ x1


def _main_body(cnt_ref, st2_ref, w_ref, b2_ref, z_ref, i_ref, l_ref, s_ref):
    j = pl.program_id(0)

    l2 = jax.lax.dot_general(
        w_ref[...], st2_ref[...], (((1,), (0,)), ((), ())),
        preferred_element_type=jnp.float32)
    l2 = l2 + b2_ref[...]

    cnt = cnt_ref[...] + j * VBLK
    bits = _threefry_bits(jax.lax.bitcast_convert_type(cnt, jnp.uint32))
    fb = (bits >> jnp.uint32(9)) | jnp.uint32(0x3F800000)
    f = jax.lax.bitcast_convert_type(fb, jnp.float32) - jnp.float32(1.0)
    u = jnp.maximum(f, _TINY)
    nlu = jnp.log(u) * jnp.float32(-1.0)
    z = nlu * jnp.exp2(-l2)

    s_ref[...] = jnp.sum(jnp.exp2(l2), axis=0, keepdims=True)[None]

    blk_zmin = jnp.min(z, axis=0, keepdims=True)
    is_min = z == blk_zmin
    blk_cnt = jnp.min(jnp.where(is_min, cnt, jnp.int32(2**31 - 1)),
                      axis=0, keepdims=True)
    l2_at = jnp.max(jnp.where(is_min, l2, -jnp.inf), axis=0, keepdims=True)
    z_ref[...] = blk_zmin[None]
    i_ref[...] = blk_cnt[None]
    l_ref[...] = l2_at[None]


def _merge_body(z_ref, i_ref, l_ref, s_ref, a_ref, lp_ref):
    z = z_ref[...]
    zmin = jnp.min(z, axis=0, keepdims=True)
    is_min = z == zmin
    bi = jax.lax.broadcasted_iota(jnp.int32, (GRID, 1, B), 0)
    kblk = jnp.min(jnp.where(is_min, bi, GRID), axis=0, keepdims=True)
    sel = bi == kblk
    cnt_win = jnp.min(jnp.where(sel, i_ref[...], jnp.int32(2**31 - 1)), axis=0)
    rv = jax.lax.broadcasted_iota(jnp.int32, (1, B), 1) * V
    a_ref[...] = cnt_win - rv
    l2_a = jnp.max(jnp.where(sel, l_ref[...], -jnp.inf), axis=0)
    lp_ref[...] = (l2_a - jnp.log2(jnp.sum(s_ref[...], axis=0))) * _LN2


def kernel(s, W, b):
    st2 = s.T * _LOG2E
    b2 = b.reshape(V, 1) * _LOG2E
    vi = jax.lax.broadcasted_iota(jnp.int32, (VBLK, B), 0)
    ri = jax.lax.broadcasted_iota(jnp.int32, (VBLK, B), 1)
    cnt0 = ri * V + vi
    part_spec = pl.BlockSpec((1, 1, B), lambda j: (j, 0, 0))
    part_shape = jax.ShapeDtypeStruct((GRID, 1, B), jnp.float32)
    zp, ip, lp_part, sp = pl.pallas_call(
        _main_body,
        grid=(GRID,),
        in_specs=[
            pl.BlockSpec((VBLK, B), lambda j: (0, 0)),
            pl.BlockSpec((64, B), lambda j: (0, 0)),
            pl.BlockSpec((VBLK, 64), lambda j: (j, 0)),
            pl.BlockSpec((VBLK, 1), lambda j: (j, 0)),
        ],
        out_specs=[part_spec, part_spec, part_spec, part_spec],
        out_shape=[
            part_shape,
            jax.ShapeDtypeStruct((GRID, 1, B), jnp.int32),
            part_shape,
            part_shape,
        ],
        compiler_params=pltpu.CompilerParams(
            dimension_semantics=("parallel",)),
    )(cnt0, st2, W, b2)
    a2, lp2 = pl.pallas_call(
        _merge_body,
        out_shape=[
            jax.ShapeDtypeStruct((1, B), jnp.int32),
            jax.ShapeDtypeStruct((1, B), jnp.float32),
        ],
    )(zp, ip, lp_part, sp)
    return a2.reshape(B), lp2.reshape(B)
